# baseline (speedup 1.0000x reference)
import functools

import jax
import jax.numpy as jnp
from jax import lax
from jax.experimental import pallas as pl
from jax.experimental.pallas import tpu as pltpu
from jax.experimental.pallas import tpu_sc as plsc

_N, _C, _H, _W = 4, 19, 512, 1024
_BH = 8
_RATIO = 0.2

_NSC = 32
_PIX = _H * _W
_PER_W = _PIX // _NSC
_CH = 4096
_NCHUNK = _PER_W // _CH
_GROUPS = _CH // 16
_ACC = _N * _C * 16


def _stage1_kernel(x_ref, s_ref, a_ref):
    x = x_ref[0]
    cur = x[0]
    idx = jnp.zeros(cur.shape, jnp.int32)
    s = cur * cur
    for c in range(1, _C):
        xc = x[c]
        gt = xc > cur
        cur = jnp.where(gt, xc, cur)
        idx = jnp.where(gt, c, idx)
        s = s + xc * xc
    s_ref[0] = s
    a_ref[0] = idx


def _stage1(prob):
    return pl.pallas_call(
        _stage1_kernel,
        grid=(_N, _H // _BH),
        in_specs=[pl.BlockSpec((1, _C, _BH, _W), lambda n, h: (n, 0, h, 0))],
        out_specs=[
            pl.BlockSpec((1, _BH, _W), lambda n, h: (n, h, 0)),
            pl.BlockSpec((1, _BH, _W), lambda n, h: (n, h, 0)),
        ],
        out_shape=[
            jax.ShapeDtypeStruct((_N, _H, _W), jnp.float32),
            jax.ShapeDtypeStruct((_N, _H, _W), jnp.int32),
        ],
    )(prob)


@functools.partial(
    pl.kernel,
    out_type=(
        jax.ShapeDtypeStruct((_NSC, _ACC), jnp.float32),
        jax.ShapeDtypeStruct((_NSC, _ACC), jnp.float32),
    ),
    mesh=plsc.VectorSubcoreMesh(core_axis_name="c", subcore_axis_name="s"),
    compiler_params=pltpu.CompilerParams(needs_layout_passes=False),
    scratch_types=[
        pltpu.VMEM((_CH,), jnp.float32),
        pltpu.VMEM((_CH,), jnp.int32),
        pltpu.VMEM((_ACC,), jnp.float32),
        pltpu.VMEM((_ACC,), jnp.float32),
    ],
)
def _stage2(s_hbm, a_hbm, cnt_hbm, sum_hbm, sbuf, abuf, cnt_v, sum_v):
    wid = lax.axis_index("c") * 16 + lax.axis_index("s")
    lane = lax.iota(jnp.int32, 16)
    ones = jnp.full((16,), 1.0, jnp.float32)
    zeros = jnp.zeros((16,), jnp.float32)

    def zbody(i, carry):
        cnt_v[pl.ds(i * 16, 16)] = zeros
        sum_v[pl.ds(i * 16, 16)] = zeros
        return carry

    lax.fori_loop(0, _ACC // 16, zbody, None)

    for chunk in range(_N * _NCHUNK):
        n = chunk // _NCHUNK
        off = n * _PIX + wid * _PER_W + (chunk % _NCHUNK) * _CH
        pltpu.sync_copy(s_hbm.at[pl.ds(off, _CH)], sbuf)
        pltpu.sync_copy(a_hbm.at[pl.ds(off, _CH)], abuf)
        base = n * (_C * 16) + lane

        def body(g, carry):
            s = sbuf[pl.ds(g * 16, 16)]
            k = abuf[pl.ds(g * 16, 16)]
            idx = base + (k << 4)
            plsc.addupdate_scatter(sum_v, [idx], s)
            plsc.addupdate_scatter(cnt_v, [idx], ones)
            return carry

        lax.fori_loop(0, _GROUPS, body, None, unroll=8)

    pltpu.sync_copy(cnt_v, cnt_hbm.at[wid])
    pltpu.sync_copy(sum_v, sum_hbm.at[wid])


def _stage3_kernel(cnt_ref, sum_ref, out_ref):
    c = jnp.sum(jnp.sum(cnt_ref[...], axis=0), axis=2)
    s = jnp.sum(jnp.sum(sum_ref[...], axis=0), axis=2)
    total = jnp.sum(c, axis=1, keepdims=True)
    denom = jnp.maximum(
        jnp.power(c, _RATIO) * jnp.power(total, 1.0 - _RATIO), 1.0
    )
    out_ref[0, 0] = -jnp.sum(s / denom) / (_N * _C)


def _stage3(cnt, ssum):
    return pl.pallas_call(
        _stage3_kernel,
        out_specs=pl.BlockSpec(memory_space=pltpu.SMEM),
        out_shape=jax.ShapeDtypeStruct((1, 1), jnp.float32),
    )(cnt, ssum)


def kernel(prob):
    s, a = _stage1(prob)
    cnt, ssum = _stage2(s.reshape(-1), a.reshape(-1))
    out = _stage3(
        cnt.reshape(_NSC, _N, _C, 16), ssum.reshape(_NSC, _N, _C, 16)
    )
    return out[0, 0]


# speedup vs baseline: 1.4442x; 1.4442x over previous
import functools

import jax
import jax.numpy as jnp
from jax import lax
from jax.experimental import pallas as pl
from jax.experimental.pallas import tpu as pltpu
from jax.experimental.pallas import tpu_sc as plsc

_N, _C, _H, _W = 4, 19, 512, 1024
_BH = 16
_RATIO = 0.2

_NSC = 32
_ROWS_W = _H // _NSC
_CROWS = 8
_NCHUNK = _ROWS_W // _CROWS
_GROUPS = _W // 16


def _stage1_kernel(x_ref, s_ref, a_ref):
    x = x_ref[0]
    cur = x[0]
    idx = jnp.zeros(cur.shape, jnp.int32)
    s = cur * cur
    for c in range(1, _C):
        xc = x[c]
        gt = xc > cur
        cur = jnp.where(gt, xc, cur)
        idx = jnp.where(gt, c, idx)
        s = s + xc * xc
    s_ref[0] = s
    a_ref[0] = idx


def _stage1(prob):
    return pl.pallas_call(
        _stage1_kernel,
        grid=(_N, _H // _BH),
        in_specs=[pl.BlockSpec((1, _C, _BH, _W), lambda n, h: (n, 0, h, 0))],
        out_specs=[
            pl.BlockSpec((1, _BH, _W), lambda n, h: (n, h, 0)),
            pl.BlockSpec((1, _BH, _W), lambda n, h: (n, h, 0)),
        ],
        out_shape=[
            jax.ShapeDtypeStruct((_N, _H, _W), jnp.float32),
            jax.ShapeDtypeStruct((_N, _H, _W), jnp.int32),
        ],
    )(prob)


@functools.partial(
    pl.kernel,
    out_type=(
        jax.ShapeDtypeStruct((_NSC, _N * _C * 16), jnp.float32),
        jax.ShapeDtypeStruct((_NSC, _N * _C * 16), jnp.float32),
    ),
    mesh=plsc.VectorSubcoreMesh(core_axis_name="c", subcore_axis_name="s"),
    compiler_params=pltpu.CompilerParams(needs_layout_passes=False),
    scratch_types=[
        pltpu.VMEM((_CROWS, _W), jnp.float32),
        pltpu.VMEM((_CROWS, _W), jnp.int32),
        pltpu.VMEM((_N * _C * 16,), jnp.float32),
        pltpu.VMEM((_N * _C * 16,), jnp.float32),
    ],
)
def _stage2(s_hbm, a_hbm, cnt_hbm, sum_hbm, sbuf, abuf, cnt_v, sum_v):
    wid = lax.axis_index("c") * 16 + lax.axis_index("s")
    lane = lax.iota(jnp.int32, 16)
    ones = jnp.full((16,), 1.0, jnp.float32)
    zeros = jnp.zeros((16,), jnp.float32)

    for i in range(_N * _C):
        cnt_v[pl.ds(i * 16, 16)] = zeros
        sum_v[pl.ds(i * 16, 16)] = zeros

    for chunk in range(_N * _NCHUNK):
        n = chunk // _NCHUNK
        row0 = wid * _ROWS_W + (chunk % _NCHUNK) * _CROWS
        pltpu.sync_copy(s_hbm.at[n, pl.ds(row0, _CROWS), :], sbuf)
        pltpu.sync_copy(a_hbm.at[n, pl.ds(row0, _CROWS), :], abuf)
        base = n * (_C * 16) + lane

        for r in range(_CROWS):

            def body(g, carry, r=r, base=base):
                s = sbuf[r, pl.ds(g * 16, 16)]
                k = abuf[r, pl.ds(g * 16, 16)]
                idx = base + (k << 4)
                plsc.addupdate_scatter(sum_v, [idx], s)
                plsc.addupdate_scatter(cnt_v, [idx], ones)
                return carry

            lax.fori_loop(0, _GROUPS, body, None, unroll=16)

    pltpu.sync_copy(cnt_v, cnt_hbm.at[wid])
    pltpu.sync_copy(sum_v, sum_hbm.at[wid])


def _stage3_kernel(cnt_ref, sum_ref, out_ref):
    c = jnp.sum(jnp.sum(cnt_ref[...], axis=0), axis=2)
    s = jnp.sum(jnp.sum(sum_ref[...], axis=0), axis=2)
    total = jnp.sum(c, axis=1, keepdims=True)
    denom = jnp.maximum(
        jnp.power(c, _RATIO) * jnp.power(total, 1.0 - _RATIO), 1.0
    )
    out_ref[0, 0] = -jnp.sum(s / denom) / (_N * _C)


def _stage3(cnt, ssum):
    return pl.pallas_call(
        _stage3_kernel,
        out_specs=pl.BlockSpec(memory_space=pltpu.SMEM),
        out_shape=jax.ShapeDtypeStruct((1, 1), jnp.float32),
    )(cnt, ssum)


def kernel(prob):
    s, a = _stage1(prob)
    cnt, ssum = _stage2(s, a)
    return _stage3(
        cnt.reshape(_NSC, _N, _C, 16), ssum.reshape(_NSC, _N, _C, 16)
    )[0, 0]


# speedup vs baseline: 1.8712x; 1.2956x over previous
import functools

import jax
import jax.numpy as jnp
from jax import lax
from jax.experimental import pallas as pl
from jax.experimental.pallas import tpu as pltpu
from jax.experimental.pallas import tpu_sc as plsc

_N, _C, _H, _W = 4, 19, 512, 1024
_BH = 16
_RATIO = 0.2

_NSC = 32
_ROWS_W = _H // _NSC
_CROWS = 8
_NCHUNK = _ROWS_W // _CROWS
_GROUPS = _W // 16
_ACC = _C * 16


def _stage1_kernel(x_ref, s_ref, a_ref):
    x = x_ref[0]
    cur = x[0]
    idx = jnp.zeros(cur.shape, jnp.int32)
    s = cur * cur
    for c in range(1, _C):
        xc = x[c]
        gt = xc > cur
        cur = jnp.where(gt, xc, cur)
        idx = jnp.where(gt, c, idx)
        s = s + xc * xc
    s_ref[...] = s
    a_ref[...] = idx


def _stage1(prob, n):
    return pl.pallas_call(
        _stage1_kernel,
        grid=(_H // _BH,),
        in_specs=[
            pl.BlockSpec((1, _C, _BH, _W), lambda h, n=n: (n, 0, h, 0))
        ],
        out_specs=[
            pl.BlockSpec((_BH, _W), lambda h: (h, 0)),
            pl.BlockSpec((_BH, _W), lambda h: (h, 0)),
        ],
        out_shape=[
            jax.ShapeDtypeStruct((_H, _W), jnp.float32),
            jax.ShapeDtypeStruct((_H, _W), jnp.int32),
        ],
    )(prob)


@functools.partial(
    pl.kernel,
    out_type=(
        jax.ShapeDtypeStruct((_NSC, _ACC), jnp.float32),
        jax.ShapeDtypeStruct((_NSC, _ACC), jnp.float32),
    ),
    mesh=plsc.VectorSubcoreMesh(core_axis_name="c", subcore_axis_name="s"),
    compiler_params=pltpu.CompilerParams(needs_layout_passes=False),
    scratch_types=[
        pltpu.VMEM((_CROWS, _W), jnp.float32),
        pltpu.VMEM((_CROWS, _W), jnp.int32),
        pltpu.VMEM((_ACC,), jnp.float32),
        pltpu.VMEM((_ACC,), jnp.float32),
    ],
)
def _stage2(s_hbm, a_hbm, cnt_hbm, sum_hbm, sbuf, abuf, cnt_v, sum_v):
    wid = lax.axis_index("c") * 16 + lax.axis_index("s")
    lane = lax.iota(jnp.int32, 16)
    ones = jnp.full((16,), 1.0, jnp.float32)
    zeros = jnp.zeros((16,), jnp.float32)

    for i in range(_C):
        cnt_v[pl.ds(i * 16, 16)] = zeros
        sum_v[pl.ds(i * 16, 16)] = zeros

    for chunk in range(_NCHUNK):
        row0 = wid * _ROWS_W + chunk * _CROWS
        pltpu.sync_copy(s_hbm.at[pl.ds(row0, _CROWS), :], sbuf)
        pltpu.sync_copy(a_hbm.at[pl.ds(row0, _CROWS), :], abuf)

        for r in range(_CROWS):

            def body(g, carry, r=r):
                s = sbuf[r, pl.ds(g * 16, 16)]
                k = abuf[r, pl.ds(g * 16, 16)]
                idx = lane + (k << 4)
                plsc.addupdate_scatter(sum_v, [idx], s)
                plsc.addupdate_scatter(cnt_v, [idx], ones)
                return carry

            lax.fori_loop(0, _GROUPS, body, None, unroll=16)

    pltpu.sync_copy(cnt_v, cnt_hbm.at[wid])
    pltpu.sync_copy(sum_v, sum_hbm.at[wid])


def _stage3_kernel(*refs):
    cnt_refs = refs[:_N]
    sum_refs = refs[_N : 2 * _N]
    out_ref = refs[2 * _N]
    c = jnp.concatenate(
        [jnp.sum(r[...], axis=0, keepdims=True) for r in cnt_refs], axis=0
    )
    s = jnp.concatenate(
        [jnp.sum(r[...], axis=0, keepdims=True) for r in sum_refs], axis=0
    )
    slot = jax.lax.broadcasted_iota(jnp.int32, (_ACC, _C), 0)
    klass = jax.lax.broadcasted_iota(jnp.int32, (_ACC, _C), 1)
    m = ((slot >> 4) == klass).astype(jnp.float32)
    hc = jnp.dot(c, m, preferred_element_type=jnp.float32)
    hs = jnp.dot(s, m, preferred_element_type=jnp.float32)
    total = jnp.sum(hc, axis=1, keepdims=True)
    denom = jnp.maximum(
        jnp.power(hc, _RATIO) * jnp.power(total, 1.0 - _RATIO), 1.0
    )
    out_ref[0, 0] = -jnp.sum(hs / denom) / (_N * _C)


def _stage3(cnts, sums):
    return pl.pallas_call(
        _stage3_kernel,
        out_specs=pl.BlockSpec(memory_space=pltpu.SMEM),
        out_shape=jax.ShapeDtypeStruct((1, 1), jnp.float32),
    )(*cnts, *sums)


def kernel(prob):
    cnts = []
    sums = []
    for n in range(_N):
        s, a = _stage1(prob, n)
        cnt, ssum = _stage2(s, a)
        cnts.append(cnt)
        sums.append(ssum)
    return _stage3(cnts, sums)[0, 0]


# speedup vs baseline: 2.3660x; 1.2644x over previous
import functools

import jax
import jax.numpy as jnp
from jax import lax
from jax.experimental import pallas as pl
from jax.experimental.pallas import tpu as pltpu
from jax.experimental.pallas import tpu_sc as plsc

_N, _C, _H, _W = 4, 19, 512, 1024
_BH = 32
_RATIO = 0.2

_NSC = 32
_ROWS_W = _H // _NSC
_CROWS = 8
_NCHUNK = _ROWS_W // _CROWS
_GROUPS = _W // 16
_ACC = _C * 16


def _stage1_kernel(x_ref, s_ref, a_ref):
    x = x_ref[0]
    cur = x[0]
    idx = jnp.zeros(cur.shape, jnp.int32)
    s = cur * cur
    for c in range(1, _C):
        xc = x[c]
        gt = xc > cur
        cur = jnp.where(gt, xc, cur)
        idx = jnp.where(gt, c, idx)
        s = s + xc * xc
    s_ref[...] = s
    a_ref[...] = idx


def _stage1(prob, n):
    return pl.pallas_call(
        _stage1_kernel,
        grid=(_H // _BH,),
        in_specs=[
            pl.BlockSpec((1, _C, _BH, _W), lambda h, n=n: (n, 0, h, 0))
        ],
        out_specs=[
            pl.BlockSpec((_BH, _W), lambda h: (h, 0)),
            pl.BlockSpec((_BH, _W), lambda h: (h, 0)),
        ],
        out_shape=[
            jax.ShapeDtypeStruct((_H, _W), jnp.float32),
            jax.ShapeDtypeStruct((_H, _W), jnp.int32),
        ],
    )(prob)


@functools.partial(
    pl.kernel,
    out_type=(
        jax.ShapeDtypeStruct((_NSC, _ACC), jnp.float32),
        jax.ShapeDtypeStruct((_NSC, _ACC), jnp.float32),
    ),
    mesh=plsc.VectorSubcoreMesh(core_axis_name="c", subcore_axis_name="s"),
    compiler_params=pltpu.CompilerParams(needs_layout_passes=False),
    scratch_types=[
        pltpu.VMEM((_CROWS, _W), jnp.float32),
        pltpu.VMEM((_CROWS, _W), jnp.int32),
        pltpu.VMEM((_ACC,), jnp.float32),
        pltpu.VMEM((_ACC,), jnp.float32),
    ],
)
def _stage2(s_hbm, a_hbm, cnt_hbm, sum_hbm, sbuf, abuf, cnt_v, sum_v):
    wid = lax.axis_index("c") * 16 + lax.axis_index("s")
    lane = lax.iota(jnp.int32, 16)
    ones = jnp.full((16,), 1.0, jnp.float32)
    zeros = jnp.zeros((16,), jnp.float32)

    for i in range(_C):
        cnt_v[pl.ds(i * 16, 16)] = zeros
        sum_v[pl.ds(i * 16, 16)] = zeros

    for chunk in range(_NCHUNK):
        row0 = wid * _ROWS_W + chunk * _CROWS
        pltpu.sync_copy(s_hbm.at[pl.ds(row0, _CROWS), :], sbuf)
        pltpu.sync_copy(a_hbm.at[pl.ds(row0, _CROWS), :], abuf)

        for r in range(_CROWS):

            def body(g, carry, r=r):
                s = sbuf[r, pl.ds(g * 16, 16)]
                k = abuf[r, pl.ds(g * 16, 16)]
                idx = lane + (k << 4)
                plsc.addupdate_scatter(sum_v, [idx], s)
                plsc.addupdate_scatter(cnt_v, [idx], ones)
                return carry

            lax.fori_loop(0, _GROUPS, body, None, unroll=16)

    pltpu.sync_copy(cnt_v, cnt_hbm.at[wid])
    pltpu.sync_copy(sum_v, sum_hbm.at[wid])


def _stage3_kernel(*refs):
    cnt_refs = refs[:_N]
    sum_refs = refs[_N : 2 * _N]
    out_ref = refs[2 * _N]
    c = jnp.concatenate(
        [jnp.sum(r[...], axis=0, keepdims=True) for r in cnt_refs], axis=0
    )
    s = jnp.concatenate(
        [jnp.sum(r[...], axis=0, keepdims=True) for r in sum_refs], axis=0
    )
    slot = jax.lax.broadcasted_iota(jnp.int32, (_ACC, _C), 0)
    klass = jax.lax.broadcasted_iota(jnp.int32, (_ACC, _C), 1)
    m = ((slot >> 4) == klass).astype(jnp.float32)
    hc = jnp.dot(c, m, preferred_element_type=jnp.float32)
    hs = jnp.dot(s, m, preferred_element_type=jnp.float32)
    total = jnp.sum(hc, axis=1, keepdims=True)
    denom = jnp.maximum(
        jnp.power(hc, _RATIO) * jnp.power(total, 1.0 - _RATIO), 1.0
    )
    out_ref[0, 0] = -jnp.sum(hs / denom) / (_N * _C)


def _stage3(cnts, sums):
    return pl.pallas_call(
        _stage3_kernel,
        out_specs=pl.BlockSpec(memory_space=pltpu.SMEM),
        out_shape=jax.ShapeDtypeStruct((1, 1), jnp.float32),
    )(*cnts, *sums)


def kernel(prob):
    cnts = []
    sums = []
    for n in range(_N):
        s, a = _stage1(prob, n)
        cnt, ssum = _stage2(s, a)
        cnts.append(cnt)
        sums.append(ssum)
    return _stage3(cnts, sums)[0, 0]


# speedup vs baseline: 2.4331x; 1.0284x over previous
import functools

import jax
import jax.numpy as jnp
from jax import lax
from jax.experimental import pallas as pl
from jax.experimental.pallas import tpu as pltpu
from jax.experimental.pallas import tpu_sc as plsc

_N, _C, _H, _W = 4, 19, 512, 1024
_BH = 64
_RATIO = 0.2

_NSC = 32
_ROWS_W = _H // _NSC
_CROWS = 8
_NCHUNK = _ROWS_W // _CROWS
_GROUPS = _W // 16
_ACC = _C * 16


def _stage1_kernel(x_ref, s_ref, a_ref):
    x = x_ref[0]
    cur = x[0]
    idx = jnp.zeros(cur.shape, jnp.int32)
    s = cur * cur
    for c in range(1, _C):
        xc = x[c]
        gt = xc > cur
        cur = jnp.where(gt, xc, cur)
        idx = jnp.where(gt, c, idx)
        s = s + xc * xc
    s_ref[...] = s
    a_ref[...] = idx


def _stage1(prob, n):
    return pl.pallas_call(
        _stage1_kernel,
        grid=(_H // _BH,),
        in_specs=[
            pl.BlockSpec((1, _C, _BH, _W), lambda h, n=n: (n, 0, h, 0))
        ],
        out_specs=[
            pl.BlockSpec((_BH, _W), lambda h: (h, 0)),
            pl.BlockSpec((_BH, _W), lambda h: (h, 0)),
        ],
        out_shape=[
            jax.ShapeDtypeStruct((_H, _W), jnp.float32),
            jax.ShapeDtypeStruct((_H, _W), jnp.int32),
        ],
    )(prob)


@functools.partial(
    pl.kernel,
    out_type=(
        jax.ShapeDtypeStruct((_NSC, _ACC), jnp.float32),
        jax.ShapeDtypeStruct((_NSC, _ACC), jnp.float32),
    ),
    mesh=plsc.VectorSubcoreMesh(core_axis_name="c", subcore_axis_name="s"),
    compiler_params=pltpu.CompilerParams(needs_layout_passes=False),
    scratch_types=[
        pltpu.VMEM((_CROWS, _W), jnp.float32),
        pltpu.VMEM((_CROWS, _W), jnp.int32),
        pltpu.VMEM((_ACC,), jnp.float32),
        pltpu.VMEM((_ACC,), jnp.float32),
    ],
)
def _stage2(s_hbm, a_hbm, cnt_hbm, sum_hbm, sbuf, abuf, cnt_v, sum_v):
    wid = lax.axis_index("c") * 16 + lax.axis_index("s")
    lane = lax.iota(jnp.int32, 16)
    ones = jnp.full((16,), 1.0, jnp.float32)
    zeros = jnp.zeros((16,), jnp.float32)

    for i in range(_C):
        cnt_v[pl.ds(i * 16, 16)] = zeros
        sum_v[pl.ds(i * 16, 16)] = zeros

    for chunk in range(_NCHUNK):
        row0 = wid * _ROWS_W + chunk * _CROWS
        pltpu.sync_copy(s_hbm.at[pl.ds(row0, _CROWS), :], sbuf)
        pltpu.sync_copy(a_hbm.at[pl.ds(row0, _CROWS), :], abuf)

        for r in range(_CROWS):

            def body(g, carry, r=r):
                s = sbuf[r, pl.ds(g * 16, 16)]
                k = abuf[r, pl.ds(g * 16, 16)]
                idx = lane + (k << 4)
                plsc.addupdate_scatter(sum_v, [idx], s)
                plsc.addupdate_scatter(cnt_v, [idx], ones)
                return carry

            lax.fori_loop(0, _GROUPS, body, None, unroll=16)

    pltpu.sync_copy(cnt_v, cnt_hbm.at[wid])
    pltpu.sync_copy(sum_v, sum_hbm.at[wid])


def _stage3_kernel(*refs):
    cnt_refs = refs[:_N]
    sum_refs = refs[_N : 2 * _N]
    out_ref = refs[2 * _N]
    c = jnp.concatenate(
        [jnp.sum(r[...], axis=0, keepdims=True) for r in cnt_refs], axis=0
    )
    s = jnp.concatenate(
        [jnp.sum(r[...], axis=0, keepdims=True) for r in sum_refs], axis=0
    )
    slot = jax.lax.broadcasted_iota(jnp.int32, (_ACC, _C), 0)
    klass = jax.lax.broadcasted_iota(jnp.int32, (_ACC, _C), 1)
    m = ((slot >> 4) == klass).astype(jnp.float32)
    hc = jnp.dot(c, m, preferred_element_type=jnp.float32)
    hs = jnp.dot(s, m, preferred_element_type=jnp.float32)
    total = jnp.sum(hc, axis=1, keepdims=True)
    denom = jnp.maximum(
        jnp.power(hc, _RATIO) * jnp.power(total, 1.0 - _RATIO), 1.0
    )
    out_ref[0, 0] = -jnp.sum(hs / denom) / (_N * _C)


def _stage3(cnts, sums):
    return pl.pallas_call(
        _stage3_kernel,
        out_specs=pl.BlockSpec(memory_space=pltpu.SMEM),
        out_shape=jax.ShapeDtypeStruct((1, 1), jnp.float32),
    )(*cnts, *sums)


def kernel(prob):
    cnts = []
    sums = []
    for n in range(_N):
        s, a = _stage1(prob, n)
        cnt, ssum = _stage2(s, a)
        cnts.append(cnt)
        sums.append(ssum)
    return _stage3(cnts, sums)[0, 0]
